# baseline (speedup 1.0000x reference)
import functools
import math

import jax
import jax.numpy as jnp
from jax import lax
from jax.experimental import pallas as pl
from jax.experimental.pallas import tpu as pltpu
from jax.experimental.pallas import tpu_sc as plsc

VOCAB_ = 1000000
N_EMBD = 64
SCALE = math.sqrt(N_EMBD)
NC, NS, LANES = 2, 16, 16
NW = NC * NS
SEQ = 200
AXIS = 4096
ABLK = AXIS // NW
PAIR_K = 499712
PAIRS = VOCAB_ - PAIR_K
P1_W = 4096
RING = 4
TB = 2


def _transpose_body(a_ref, b_ref, out_ref):
    out_ref[:, 0:64] = jnp.swapaxes(a_ref[...], 0, 1) * SCALE
    out_ref[:, 64:128] = jnp.swapaxes(b_ref[...], 0, 1) * SCALE


def _make_lut_pairs(lutT):
    n_blk = (PAIRS + P1_W - 1) // P1_W
    return pl.pallas_call(
        _transpose_body,
        grid=(n_blk,),
        in_specs=[
            pl.BlockSpec((64, P1_W), lambda j: (0, j)),
            pl.BlockSpec((64, P1_W), lambda j: (0, j + PAIR_K // P1_W)),
        ],
        out_specs=pl.BlockSpec((P1_W, 128), lambda j: (j, 0)),
        out_shape=jax.ShapeDtypeStruct((PAIRS, 128), jnp.float32),
    )(lutT, lutT)


@jax.jit
def _emb(lut, x):
    lutT = lut.T
    xT = x.T
    lutP = _make_lut_pairs(lutT)

    mesh = plsc.VectorSubcoreMesh(core_axis_name="c", subcore_axis_name="s")

    @functools.partial(
        pl.kernel,
        mesh=mesh,
        compiler_params=pltpu.CompilerParams(
            use_tc_tiling_on_sc=True, needs_layout_passes=False
        ),
        out_type=jax.ShapeDtypeStruct((SEQ, 8, NW, 8, 128), jnp.float32),
        scratch_types=[
            pltpu.VMEM((SEQ, 128), jnp.int32),
            pltpu.VMEM((8, 128), jnp.int32),
            *[pltpu.VMEM((128, 128), jnp.float32) for _ in range(RING)],
            *[pltpu.VMEM((8, 8, 128), jnp.float32) for _ in range(TB)],
            *[pltpu.SemaphoreType.DMA for _ in range(RING + TB)],
        ],
    )
    def k2(lutP_hbm, xT_hbm, out_hbm, xv, idxr,
           g0, g1, g2, g3, t0, t1, sg0, sg1, sg2, sg3, ss0, ss1):
        wid = lax.axis_index("s") * NC + lax.axis_index("c")
        gbufs = (g0, g1, g2, g3)
        gsems = (sg0, sg1, sg2, sg3)
        tbufs = (t0, t1)
        ssems = (ss0, ss1)
        pltpu.sync_copy(xT_hbm.at[:, pl.ds(wid * ABLK, ABLK)], xv)

        def prep_fire(p, r):
            for l0 in range(0, 128, LANES):
                sl = pl.ds(l0, LANES)
                v = xv[p, sl]
                hi = (v >= PAIR_K).astype(jnp.int32)
                idxr[r, sl] = v - hi * PAIR_K
            pltpu.async_copy(lutP_hbm.at[idxr.at[r]], gbufs[r], gsems[r])

        def wait_g(r):
            pltpu.make_async_copy(
                lutP_hbm.at[idxr.at[r]], gbufs[r], gsems[r]
            ).wait()

        def fire_s(p, t):
            pltpu.async_copy(
                tbufs[t], out_hbm.at[p, pl.ds(0, 8), wid], ssems[t]
            )

        def wait_s(p, t):
            pltpu.make_async_copy(
                tbufs[t], out_hbm.at[p, pl.ds(0, 8), wid], ssems[t]
            ).wait()

        for r0 in range(RING - 1):
            prep_fire(r0, r0)

        rows = [
            jnp.arange(LANES, dtype=jnp.int32) + l0
            for l0 in range(0, 128, LANES)
        ]

        def p_body(g, carry):
            for r in range(RING):
                p = RING * g + r

                @pl.when(p + RING - 1 < SEQ)
                def _():
                    prep_fire(p + RING - 1, (r + RING - 1) % RING)

                wait_g(r)
                t = r % TB

                @pl.when(p >= TB)
                def _():
                    wait_s(p, t)

                for li in range(8):
                    par = (
                        xv[p, pl.ds(li * LANES, LANES)] >= PAIR_K
                    ).astype(jnp.int32) * N_EMBD
                    for e in range(N_EMBD):
                        v = plsc.load_gather(gbufs[r], [rows[li], par + e])
                        tbufs[t][e // 8, e % 8, pl.ds(li * LANES, LANES)] = v
                fire_s(p, t)
            return carry

        lax.fori_loop(0, SEQ // RING, p_body, 0)
        wait_s(SEQ - 2, 0)
        wait_s(SEQ - 1, 1)

    out5 = k2(lutP, xT)
    return out5.transpose(2, 4, 0, 1, 3).reshape(AXIS, SEQ, N_EMBD)


def kernel(x, lut):
    return _emb(lut, x)


# speedup vs baseline: 1.2156x; 1.2156x over previous
import functools
import math

import jax
import jax.numpy as jnp
from jax import lax
from jax.experimental import pallas as pl
from jax.experimental.pallas import tpu as pltpu
from jax.experimental.pallas import tpu_sc as plsc

N_EMBD = 64
SCALE = math.sqrt(N_EMBD)

NC = 2
NS = 16
NW = NC * NS
SEQ = 200
SPLITS = ((0, 104), (104, 96))
R = 4
LANES = 16


@jax.jit
def _emb_lookup(lut, x):
    n_rows = x.shape[0]
    rows_per_w = n_rows // NW
    n_chunks = rows_per_w // R
    mesh = plsc.VectorSubcoreMesh(core_axis_name="c", subcore_axis_name="s")

    @functools.partial(
        pl.kernel,
        mesh=mesh,
        compiler_params=pltpu.CompilerParams(use_tc_tiling_on_sc=False),
        out_type=jax.ShapeDtypeStruct((n_rows, SEQ, N_EMBD), jnp.float32),
        scratch_types=[
            pltpu.VMEM((rows_per_w, SEQ), jnp.int32),
            pltpu.VMEM((R, SEQ, N_EMBD), jnp.float32),
            pltpu.SemaphoreType.DMA,
        ],
    )
    def k(lut_hbm, x_hbm, out_hbm, idx_v, rows_v, sem):
        wid = lax.axis_index("s") * NC + lax.axis_index("c")
        row0 = wid * rows_per_w
        pltpu.sync_copy(x_hbm.at[pl.ds(row0, rows_per_w)], idx_v)

        def chunk_body(g, carry):
            cps = []
            for r in range(R):
                for off, width in SPLITS:
                    cps.append(
                        pltpu.async_copy(
                            lut_hbm.at[idx_v.at[g * R + r, pl.ds(off, width)]],
                            rows_v.at[r, pl.ds(off, width)],
                            sem,
                        )
                    )
            for cp in cps:
                cp.wait()

            def scale_pos(p, carry2):
                for r in range(R):
                    for j in range(N_EMBD // LANES):
                        sl = pl.ds(j * LANES, LANES)
                        rows_v[r, p, sl] = rows_v[r, p, sl] * SCALE
                return carry2

            lax.fori_loop(0, SEQ, scale_pos, 0)
            pltpu.sync_copy(rows_v, out_hbm.at[pl.ds(row0 + g * R, R)])
            return carry

        lax.fori_loop(0, n_chunks, chunk_body, 0)

    return k(lut, x)


def kernel(x, lut):
    return _emb_lookup(lut, x)
